# baseline (speedup 1.0000x reference)
import functools

import numpy as np

import jax
import jax.numpy as jnp
from jax.experimental import pallas as pl
from jax.experimental.pallas import tpu as pltpu

N = 50000
NIN = 6
NH = 64
NOUT = 4
DEPTH = 3
RATIOS = [0.5, 0.5, 0.5]
RADII = [0.2, 0.4, 0.8]
MAXNB = 32


def _mlp_in_kernel(x_ref, w0_ref, b0_ref, w1_ref, b1_ref, o_ref):
    h = jnp.maximum(x_ref[...] @ w0_ref[...] + b0_ref[...], 0.0)
    o_ref[...] = jnp.maximum(h @ w1_ref[...] + b1_ref[...], 0.0)


def _mlp_in(x, W0, b0, W1, b1):
    n = x.shape[0]
    blk = 2000
    grid = (n + blk - 1) // blk
    return pl.pallas_call(
        _mlp_in_kernel,
        grid=(grid,),
        in_specs=[
            pl.BlockSpec((blk, NIN), lambda i: (i, 0)),
            pl.BlockSpec((NIN, 64), lambda i: (0, 0)),
            pl.BlockSpec((64,), lambda i: (0,)),
            pl.BlockSpec((64, NH), lambda i: (0, 0)),
            pl.BlockSpec((NH,), lambda i: (0,)),
        ],
        out_specs=pl.BlockSpec((blk, NH), lambda i: (i, 0)),
        out_shape=jax.ShapeDtypeStruct((n, NH), jnp.float32),
    )(x, W0, b0, W1, b1)


def _fps_kernel(n, n_keep, px_ref, py_ref, pz_ref, out_ref, dist_ref):
    R = px_ref.shape[0]
    flat = (jax.lax.broadcasted_iota(jnp.int32, (R, 128), 0) * 128
            + jax.lax.broadcasted_iota(jnp.int32, (R, 128), 1))
    lane = jax.lax.broadcasted_iota(jnp.int32, (1, 128), 1)
    dist_ref[...] = jnp.where(flat < n, jnp.inf, -jnp.inf)
    out_ref[0:1, :] = jnp.zeros((1, 1), jnp.int32)

    def body(i, last):
        r = last // 128
        c = last - r * 128
        lm = (lane == c)
        sx = jnp.sum(jnp.where(lm, px_ref[pl.ds(r, 1), :], 0.0))
        sy = jnp.sum(jnp.where(lm, py_ref[pl.ds(r, 1), :], 0.0))
        sz = jnp.sum(jnp.where(lm, pz_ref[pl.ds(r, 1), :], 0.0))
        dx = px_ref[...] - sx
        dy = py_ref[...] - sy
        dz = pz_ref[...] - sz
        d = (dx * dx + dy * dy) + dz * dz
        nd = jnp.minimum(dist_ref[...], d)
        dist_ref[...] = nd
        m = jnp.max(nd)
        idx = jnp.min(jnp.where(nd == m, flat, jnp.int32(2 ** 30)))
        out_ref[pl.ds(i, 1), :] = idx.reshape(1, 1)
        return idx

    jax.lax.fori_loop(1, n_keep, body, jnp.int32(0))


def _fps_idx(pos, n_keep):
    n = pos.shape[0]
    R = (n + 127) // 128
    Kr = (n_keep + 127) // 128
    pad = R * 128 - n
    p = jnp.pad(pos, ((0, pad), (0, 0)))
    px = p[:, 0].reshape(R, 128)
    py = p[:, 1].reshape(R, 128)
    pz = p[:, 2].reshape(R, 128)
    out = pl.pallas_call(
        functools.partial(_fps_kernel, n, n_keep),
        grid=(1,),
        in_specs=[pl.BlockSpec((R, 128), lambda i: (0, 0))] * 3,
        out_specs=pl.BlockSpec((Kr * 128, 1), lambda i: (0, 0)),
        out_shape=jax.ShapeDtypeStruct((Kr * 128, 1), jnp.int32),
        scratch_shapes=[pltpu.VMEM((R, 128), jnp.float32)],
    )(px, py, pz)
    return out.reshape(-1)[:n_keep]


def _nb_kernel(n_s, R, C, r2,
               psx_ref, psy_ref, psz_ref, pdx_ref, pdy_ref, pdz_ref,
               idx_ref, d2o_ref,
               d2_ref, cm_ref, mb_ref, cqb_ref, ncmb_ref):
    DB = 128
    CP = cm_ref.shape[1]
    lane1 = jax.lax.broadcasted_iota(jnp.int32, (1, 128), 1)
    pdx = pdx_ref[...]
    pdy = pdy_ref[...]
    pdz = pdz_ref[...]

    def d2_body(t, _):
        sx = psx_ref[pl.ds(t, 1), :]
        sy = psy_ref[pl.ds(t, 1), :]
        sz = psz_ref[pl.ds(t, 1), :]
        dx = pdx - sx
        dy = pdy - sy
        dz = pdz - sz
        d2 = (dx * dx + dy * dy) + dz * dz
        ok = (d2 <= r2) & (t * 128 + lane1 < n_s)
        d2_ref[:, pl.ds(t * 128, 128)] = jnp.where(ok, d2, jnp.inf)
        return 0

    jax.lax.fori_loop(0, R, d2_body, 0)

    cm_ref[...] = jnp.full(cm_ref.shape, jnp.inf, jnp.float32)
    for c in range(C):
        ch = d2_ref[:, c * 128:(c + 1) * 128]
        cm_ref[:, c:c + 1] = jnp.min(ch, axis=1, keepdims=True)

    laneC = jax.lax.broadcasted_iota(jnp.int32, (DB, CP), 1)

    def round_body(k, _):
        cm = cm_ref[...]
        m = jnp.min(cm, axis=1, keepdims=True)
        cq = jnp.min(jnp.where(cm == m, laneC, jnp.int32(2 ** 30)), axis=1,
                     keepdims=True)
        mb_ref[...] = m
        cqb_ref[...] = cq

        for q in range(DB):
            m_q = mb_ref[q:q + 1, 0:1][0, 0]
            c = cqb_ref[q:q + 1, 0:1][0, 0]
            chunk = d2_ref[q:q + 1, pl.ds(c * 128, 128)]
            fl = c * 128 + lane1
            sel = jnp.min(jnp.where(chunk == m_q, fl, jnp.int32(2 ** 30)))
            masked = jnp.where(fl == sel, jnp.inf, chunk)
            d2_ref[q:q + 1, pl.ds(c * 128, 128)] = masked
            ncmb_ref[q:q + 1, 0:1] = jnp.min(masked).reshape(1, 1)
            idx_w = jnp.where(m_q == jnp.inf, 0, sel)
            idx_ref[pl.ds(q * MAXNB + k, 1), 0:1] = idx_w.reshape(1, 1)
            d2o_ref[pl.ds(q * MAXNB + k, 1), 0:1] = m_q.reshape(1, 1)
        cm_ref[...] = jnp.where(laneC == cqb_ref[...], ncmb_ref[...],
                                cm_ref[...])
        return 0

    jax.lax.fori_loop(0, MAXNB, round_body, 0)


def _radius_graph(pos_src, pos_dst, r, max_nb):
    n_s = pos_src.shape[0]
    n_d = pos_dst.shape[0]
    R = (n_s + 127) // 128
    S = R * 128
    G = (n_d + 127) // 128
    NDP = G * 128
    C = R
    CP = ((C + 127) // 128) * 128
    r2 = np.float32(r * r)

    ps = jnp.pad(pos_src, ((0, S - n_s), (0, 0)))
    psx = ps[:, 0].reshape(R, 128)
    psy = ps[:, 1].reshape(R, 128)
    psz = ps[:, 2].reshape(R, 128)
    pd = jnp.pad(pos_dst, ((0, NDP - n_d), (0, 0)), constant_values=1e9)
    pdx = pd[:, 0:1]
    pdy = pd[:, 1:2]
    pdz = pd[:, 2:3]

    idx, d2o = pl.pallas_call(
        functools.partial(_nb_kernel, n_s, R, C, r2),
        grid=(G,),
        in_specs=[pl.BlockSpec((R, 128), lambda i: (0, 0))] * 3
        + [pl.BlockSpec((128, 1), lambda i: (i, 0))] * 3,
        out_specs=[pl.BlockSpec((128 * MAXNB, 1), lambda i: (i, 0))] * 2,
        out_shape=[jax.ShapeDtypeStruct((NDP * MAXNB, 1), jnp.int32),
                   jax.ShapeDtypeStruct((NDP * MAXNB, 1), jnp.float32)],
        scratch_shapes=[pltpu.VMEM((128, S), jnp.float32),
                        pltpu.VMEM((128, CP), jnp.float32),
                        pltpu.VMEM((128, 1), jnp.float32),
                        pltpu.VMEM((128, 1), jnp.int32),
                        pltpu.VMEM((128, 1), jnp.float32)],
    )(psx, psy, psz, pdx, pdy, pdz)
    idx = idx.reshape(NDP, MAXNB)[:n_d]
    d2v = d2o.reshape(NDP, MAXNB)[:n_d]
    valid = d2v != jnp.inf
    return idx, valid


def _ppf(pos_i, pos_j, n_i, n_j):
    d = pos_j - pos_i
    dn = jnp.linalg.norm(d, axis=-1, keepdims=True)

    def ang(a, b):
        cross = jnp.linalg.norm(jnp.cross(a, b), axis=-1)
        dot = jnp.sum(a * b, axis=-1)
        return jnp.arctan2(cross, dot)[..., None]

    return jnp.concatenate([dn, ang(n_i, d), ang(n_j, d), ang(n_i, n_j)], axis=-1)


def kernel(x, pos, norm, batch,
           W_in0, b_in0, W_in1, b_in1,
           Wf0, bf0, Ws0, bs0, Wf1, bf1, Ws1, bs1, Wf2, bf2, Ws2, bs2,
           Wg0, bg0, Wg1, bg1, W_out0, b_out0, W_out1, b_out1):
    h = _mlp_in(x, W_in0, b_in0, W_in1, b_in1)

    Wfs = [Wf0, Wf1, Wf2]; bfs = [bf0, bf1, bf2]
    Wss = [Ws0, Ws1, Ws2]; bss = [bs0, bs1, bs2]

    cur_x, cur_pos, cur_norm = h, pos, norm
    n_cur = N
    for lvl in range(DEPTH):
        n_keep = int(n_cur * RATIOS[lvl])
        sel = _fps_idx(cur_pos, n_keep)
        pos_d = cur_pos[sel]
        norm_d = cur_norm[sel]
        x_d = cur_x[sel]
        nb_idx, valid = _radius_graph(cur_pos, pos_d, RADII[lvl], MAXNB)
        x_j = cur_x[nb_idx]
        pos_j = cur_pos[nb_idx]
        n_j = cur_norm[nb_idx]
        ppf = _ppf(pos_d[:, None, :], pos_j, norm_d[:, None, :], n_j)
        x_i = jnp.broadcast_to(x_d[:, None, :], x_j.shape)
        z = jnp.concatenate([x_i, x_j, ppf], axis=-1)
        msg = jax.nn.sigmoid(z @ Wfs[lvl] + bfs[lvl]) * jax.nn.softplus(z @ Wss[lvl] + bss[lvl])
        msg = jnp.where(valid[..., None], msg, 0.0)
        cnt = jnp.maximum(jnp.sum(valid, axis=-1, keepdims=True).astype(jnp.float32), 1.0)
        agg = jnp.sum(msg, axis=1) / cnt
        cur_x = x_d + agg
        cur_pos = pos_d
        cur_norm = norm_d
        n_cur = n_keep

    gate = jax.nn.relu(cur_x @ Wg0 + bg0) @ Wg1 + bg1
    attn = jax.nn.softmax(gate, axis=0)
    pooled = jnp.sum(attn * cur_x, axis=0, keepdims=True)

    out = jax.nn.relu(pooled @ W_out0 + b_out0) @ W_out1 + b_out1
    return out


# speedup vs baseline: 5.4388x; 5.4388x over previous
import functools

import numpy as np

import jax
import jax.numpy as jnp
from jax.experimental import pallas as pl
from jax.experimental.pallas import tpu as pltpu

N = 50000
NIN = 6
NH = 64
NOUT = 4
DEPTH = 3
RATIOS = [0.5, 0.5, 0.5]
RADII = [0.2, 0.4, 0.8]
MAXNB = 32
BIG = 2 ** 30


def _mlp_in_kernel(x_ref, w0_ref, b0_ref, w1_ref, b1_ref, o_ref):
    h = jnp.maximum(x_ref[...] @ w0_ref[...] + b0_ref[...], 0.0)
    o_ref[...] = jnp.maximum(h @ w1_ref[...] + b1_ref[...], 0.0)


def _mlp_in(x, W0, b0, W1, b1):
    n = x.shape[0]
    blk = 2000
    grid = (n + blk - 1) // blk
    return pl.pallas_call(
        _mlp_in_kernel,
        grid=(grid,),
        in_specs=[
            pl.BlockSpec((blk, NIN), lambda i: (i, 0)),
            pl.BlockSpec((NIN, 64), lambda i: (0, 0)),
            pl.BlockSpec((64,), lambda i: (0,)),
            pl.BlockSpec((64, NH), lambda i: (0, 0)),
            pl.BlockSpec((NH,), lambda i: (0,)),
        ],
        out_specs=pl.BlockSpec((blk, NH), lambda i: (i, 0)),
        out_shape=jax.ShapeDtypeStruct((n, NH), jnp.float32),
    )(x, W0, b0, W1, b1)


def _fps_kernel(n, n_keep, px_ref, py_ref, pz_ref, out_ref, dist_ref):
    R = px_ref.shape[0]
    flat = (jax.lax.broadcasted_iota(jnp.int32, (R, 128), 0) * 128
            + jax.lax.broadcasted_iota(jnp.int32, (R, 128), 1))
    lane = jax.lax.broadcasted_iota(jnp.int32, (1, 128), 1)
    dist_ref[...] = jnp.where(flat < n, jnp.inf, -jnp.inf)
    out_ref[0:1, :] = jnp.zeros((1, 1), jnp.int32)

    def body(i, last):
        r = last // 128
        c = last - r * 128
        lm = (lane == c)
        sx = jnp.sum(jnp.where(lm, px_ref[pl.ds(r, 1), :], 0.0))
        sy = jnp.sum(jnp.where(lm, py_ref[pl.ds(r, 1), :], 0.0))
        sz = jnp.sum(jnp.where(lm, pz_ref[pl.ds(r, 1), :], 0.0))
        dx = px_ref[...] - sx
        dy = py_ref[...] - sy
        dz = pz_ref[...] - sz
        d = (dx * dx + dy * dy) + dz * dz
        nd = jnp.minimum(dist_ref[...], d)
        dist_ref[...] = nd
        m = jnp.max(nd)
        idx = jnp.min(jnp.where(nd == m, flat, jnp.int32(2 ** 30)))
        out_ref[pl.ds(i, 1), :] = idx.reshape(1, 1)
        return idx

    jax.lax.fori_loop(1, n_keep, body, jnp.int32(0))


def _fps_idx(pos, n_keep):
    n = pos.shape[0]
    R = (n + 127) // 128
    Kr = (n_keep + 127) // 128
    pad = R * 128 - n
    p = jnp.pad(pos, ((0, pad), (0, 0)))
    px = p[:, 0].reshape(R, 128)
    py = p[:, 1].reshape(R, 128)
    pz = p[:, 2].reshape(R, 128)
    out = pl.pallas_call(
        functools.partial(_fps_kernel, n, n_keep),
        grid=(1,),
        in_specs=[pl.BlockSpec((R, 128), lambda i: (0, 0))] * 3,
        out_specs=pl.BlockSpec((Kr * 128, 1), lambda i: (0, 0)),
        out_shape=jax.ShapeDtypeStruct((Kr * 128, 1), jnp.int32),
        scratch_shapes=[pltpu.VMEM((R, 128), jnp.float32)],
    )(px, py, pz)
    return out.reshape(-1)[:n_keep]


def _nb_kernel(n_s, R, r2,
               psx_ref, psy_ref, psz_ref, pdx_ref, pdy_ref, pdz_ref,
               idx_ref, d2o_ref):
    DB = idx_ref.shape[0]
    lane1 = jax.lax.broadcasted_iota(jnp.int32, (1, 128), 1)
    laneD = jax.lax.broadcasted_iota(jnp.int32, (DB, 128), 1)
    laneK = jax.lax.broadcasted_iota(jnp.int32, (DB, MAXNB), 1)
    pdx = pdx_ref[...]
    pdy = pdy_ref[...]
    pdz = pdz_ref[...]
    inf = jnp.float32(jnp.inf)

    def init_f():
        return jnp.full((DB, 128), inf, jnp.float32)

    def init_i():
        return jnp.zeros((DB, 128), jnp.int32)

    def body(t, carry):
        M1, M2, M3, M4, A1, A2, A3, A4 = carry
        sx = psx_ref[pl.ds(t, 1), :]
        sy = psy_ref[pl.ds(t, 1), :]
        sz = psz_ref[pl.ds(t, 1), :]
        dx = pdx - sx
        dy = pdy - sy
        dz = pdz - sz
        d2 = (dx * dx + dy * dy) + dz * dz
        ok = (d2 <= r2) & (t * 128 + lane1 < n_s)
        v = jnp.where(ok, d2, inf)
        vi = jnp.zeros((DB, 128), jnp.int32) + t
        lt = v < M1
        M1, v2 = jnp.where(lt, v, M1), jnp.where(lt, M1, v)
        A1, vi2 = jnp.where(lt, vi, A1), jnp.where(lt, A1, vi)
        lt = v2 < M2
        M2, v3 = jnp.where(lt, v2, M2), jnp.where(lt, M2, v2)
        A2, vi3 = jnp.where(lt, vi2, A2), jnp.where(lt, A2, vi2)
        lt = v3 < M3
        M3, v4 = jnp.where(lt, v3, M3), jnp.where(lt, M3, v3)
        A3, vi4 = jnp.where(lt, vi3, A3), jnp.where(lt, A3, vi3)
        lt = v4 < M4
        M4 = jnp.where(lt, v4, M4)
        A4 = jnp.where(lt, vi4, A4)
        return (M1, M2, M3, M4, A1, A2, A3, A4)

    M1, M2, M3, M4, A1, A2, A3, A4 = jax.lax.fori_loop(
        0, R, body, (init_f(), init_f(), init_f(), init_f(), init_i(),
                     init_i(), init_i(), init_i()))

    out_i = jnp.zeros((DB, MAXNB), jnp.int32)
    out_d = jnp.zeros((DB, MAXNB), jnp.float32)
    for k in range(MAXNB):
        m = jnp.min(M1, axis=1, keepdims=True)
        mask = M1 == m
        flv = A1 * 128 + laneD
        sel = jnp.min(jnp.where(mask, flv, BIG), axis=1, keepdims=True)
        oh = mask & (flv == sel)
        M1 = jnp.where(oh, M2, M1)
        A1 = jnp.where(oh, A2, A1)
        M2 = jnp.where(oh, M3, M2)
        A2 = jnp.where(oh, A3, A2)
        M3 = jnp.where(oh, M4, M3)
        A3 = jnp.where(oh, A4, A3)
        M4 = jnp.where(oh, inf, M4)
        out_i = jnp.where(laneK == k, sel, out_i)
        out_d = jnp.where(laneK == k, m, out_d)
    out_i = jnp.where(out_d == inf, 0, out_i)
    idx_ref[...] = out_i
    d2o_ref[...] = out_d


def _radius_graph(pos_src, pos_dst, r, max_nb):
    n_s = pos_src.shape[0]
    n_d = pos_dst.shape[0]
    R = (n_s + 127) // 128
    S = R * 128
    DB = 32
    G = (n_d + DB - 1) // DB
    NDP = G * DB
    r2 = np.float32(r * r)

    ps = jnp.pad(pos_src, ((0, S - n_s), (0, 0)))
    psx = ps[:, 0].reshape(R, 128)
    psy = ps[:, 1].reshape(R, 128)
    psz = ps[:, 2].reshape(R, 128)
    pd = jnp.pad(pos_dst, ((0, NDP - n_d), (0, 0)), constant_values=1e9)
    pdx = pd[:, 0:1]
    pdy = pd[:, 1:2]
    pdz = pd[:, 2:3]

    idx, d2o = pl.pallas_call(
        functools.partial(_nb_kernel, n_s, R, r2),
        grid=(G,),
        in_specs=[pl.BlockSpec((R, 128), lambda i: (0, 0))] * 3
        + [pl.BlockSpec((DB, 1), lambda i: (i, 0))] * 3,
        out_specs=[pl.BlockSpec((DB, MAXNB), lambda i: (i, 0))] * 2,
        out_shape=[jax.ShapeDtypeStruct((NDP, MAXNB), jnp.int32),
                   jax.ShapeDtypeStruct((NDP, MAXNB), jnp.float32)],
    )(psx, psy, psz, pdx, pdy, pdz)
    idx = idx[:n_d]
    d2v = d2o[:n_d]
    valid = d2v != jnp.inf
    return idx, valid


def _ppf(pos_i, pos_j, n_i, n_j):
    d = pos_j - pos_i
    dn = jnp.linalg.norm(d, axis=-1, keepdims=True)

    def ang(a, b):
        cross = jnp.linalg.norm(jnp.cross(a, b), axis=-1)
        dot = jnp.sum(a * b, axis=-1)
        return jnp.arctan2(cross, dot)[..., None]

    return jnp.concatenate([dn, ang(n_i, d), ang(n_j, d), ang(n_i, n_j)], axis=-1)


def kernel(x, pos, norm, batch,
           W_in0, b_in0, W_in1, b_in1,
           Wf0, bf0, Ws0, bs0, Wf1, bf1, Ws1, bs1, Wf2, bf2, Ws2, bs2,
           Wg0, bg0, Wg1, bg1, W_out0, b_out0, W_out1, b_out1):
    h = _mlp_in(x, W_in0, b_in0, W_in1, b_in1)

    Wfs = [Wf0, Wf1, Wf2]; bfs = [bf0, bf1, bf2]
    Wss = [Ws0, Ws1, Ws2]; bss = [bs0, bs1, bs2]

    cur_x, cur_pos, cur_norm = h, pos, norm
    n_cur = N
    for lvl in range(DEPTH):
        n_keep = int(n_cur * RATIOS[lvl])
        sel = _fps_idx(cur_pos, n_keep)
        pos_d = cur_pos[sel]
        norm_d = cur_norm[sel]
        x_d = cur_x[sel]
        nb_idx, valid = _radius_graph(cur_pos, pos_d, RADII[lvl], MAXNB)
        x_j = cur_x[nb_idx]
        pos_j = cur_pos[nb_idx]
        n_j = cur_norm[nb_idx]
        ppf = _ppf(pos_d[:, None, :], pos_j, norm_d[:, None, :], n_j)
        x_i = jnp.broadcast_to(x_d[:, None, :], x_j.shape)
        z = jnp.concatenate([x_i, x_j, ppf], axis=-1)
        msg = jax.nn.sigmoid(z @ Wfs[lvl] + bfs[lvl]) * jax.nn.softplus(z @ Wss[lvl] + bss[lvl])
        msg = jnp.where(valid[..., None], msg, 0.0)
        cnt = jnp.maximum(jnp.sum(valid, axis=-1, keepdims=True).astype(jnp.float32), 1.0)
        agg = jnp.sum(msg, axis=1) / cnt
        cur_x = x_d + agg
        cur_pos = pos_d
        cur_norm = norm_d
        n_cur = n_keep

    gate = jax.nn.relu(cur_x @ Wg0 + bg0) @ Wg1 + bg1
    attn = jax.nn.softmax(gate, axis=0)
    pooled = jnp.sum(attn * cur_x, axis=0, keepdims=True)

    out = jax.nn.relu(pooled @ W_out0 + b_out0) @ W_out1 + b_out1
    return out
